# Initial kernel scaffold; baseline (speedup 1.0000x reference)
import jax
import jax.numpy as jnp
from jax.experimental import pallas as pl


def kernel(x_product, x_customer, edge_index_pp, edge_index_pc, W1l, b1l, W1r, W2l, b2l, W2r, W3l, b3l, W3r, Wlin, blin):
    raise NotImplementedError("write your pallas kernel here")


# speedup vs baseline: 8.2287x; 8.2287x over previous
import functools

import jax
import jax.numpy as jnp
from jax import lax
from jax.experimental import pallas as pl
from jax.experimental.pallas import tpu as pltpu
from jax.experimental.pallas import tpu_sc as plsc

N = 10000
D = 128
WPAD = 144
K = 125
IB = 8
NSUB = 16
ROWS_PER_TILE = N // NSUB
R = 1000


def _make_sc_segsum(n_chunks):
    mesh = plsc.VectorSubcoreMesh(core_axis_name="c", subcore_axis_name="s")

    ib = IB
    n_blocks = n_chunks // ib
    n_super = n_blocks // 2

    @functools.partial(
        pl.kernel,
        out_type=(
            jax.ShapeDtypeStruct((N, WPAD), jnp.float32),
            jax.ShapeDtypeStruct((N, WPAD), jnp.float32),
        ),
        mesh=mesh,
        scratch_types=[
            pltpu.VMEM((2, ib, K), jnp.int32),
            pltpu.VMEM((2, ib, K), jnp.int32),
            pltpu.VMEM((K, WPAD), jnp.float32),
            pltpu.VMEM((K, WPAD), jnp.float32),
            pltpu.VMEM_SHARED((N, WPAD), jnp.float32),
            pltpu.SemaphoreType.DMA,
            pltpu.SemaphoreType.DMA,
            pltpu.SemaphoreType.DMA,
        ],
        compiler_params=pltpu.CompilerParams(use_tc_tiling_on_sc=False),
    )
    def sc_segsum(table_a, table_b, src_a, dst_a, src_b, dst_b, zeros_hbm,
                  out_a, out_b, src_v, dst_v, rows0, rows1, acc,
                  sem_i, sem_g0, sem_g1):
        c = lax.axis_index("c")
        s = lax.axis_index("s")
        r0 = s * ROWS_PER_TILE
        rows = (rows0, rows1)
        sem_g = (sem_g0, sem_g1)

        pltpu.sync_copy(zeros_hbm, acc.at[pl.ds(r0, ROWS_PER_TILE)])
        plsc.subcore_barrier()

        def process(src_hbm, dst_hbm, table):
            def load_idx(b, buf):
                pltpu.async_copy(src_hbm.at[s, b], src_v.at[buf], sem_i)
                pltpu.async_copy(dst_hbm.at[s, b], dst_v.at[buf], sem_i)

            def wait_idx(buf):
                pltpu.make_async_copy(src_hbm.at[s, 0], src_v.at[buf], sem_i).wait()
                pltpu.make_async_copy(dst_hbm.at[s, 0], dst_v.at[buf], sem_i).wait()

            def fire_gather(pb, jj, q):
                pltpu.async_copy(table.at[src_v.at[pb, jj]], rows[q], sem_g[q])

            def wait_gather(pb, jj, q):
                pltpu.make_async_copy(table.at[src_v.at[pb, jj]], rows[q],
                                      sem_g[q]).wait()

            load_idx(0, 0)
            wait_idx(0)
            fire_gather(0, 0, 0)

            def super_body(g, carry):
                for pb in range(2):
                    if pb == 0:
                        load_idx(2 * g + 1, 1)
                    else:
                        @pl.when(g + 1 < n_super)
                        def _():
                            load_idx(2 * g + 2, 0)
                    for jj in range(ib):
                        q = jj % 2
                        wait_gather(pb, jj, q)
                        if jj < ib - 1:
                            fire_gather(pb, jj + 1, 1 - q)
                        elif pb == 0:
                            wait_idx(1)
                            fire_gather(1, 0, 1 - q)
                        else:
                            @pl.when(g + 1 < n_super)
                            def _():
                                wait_idx(0)
                                fire_gather(0, 0, 1 - q)
                        pltpu.sync_copy(rows[q], acc.at[dst_v.at[pb, jj]],
                                        add=True)
                return carry

            lax.fori_loop(0, n_super, super_body, 0)

        @pl.when(c == 0)
        def _():
            process(src_a, dst_a, table_a)

        @pl.when(c == 1)
        def _():
            process(src_b, dst_b, table_b)

        plsc.subcore_barrier()

        @pl.when(c == 0)
        def _():
            pltpu.sync_copy(acc.at[pl.ds(r0, ROWS_PER_TILE)],
                            out_a.at[pl.ds(r0, ROWS_PER_TILE)])

        @pl.when(c == 1)
        def _():
            pltpu.sync_copy(acc.at[pl.ds(r0, ROWS_PER_TILE)],
                            out_b.at[pl.ds(r0, ROWS_PER_TILE)])

    return sc_segsum


def _pad_table(h):
    rows = h.shape[0]
    return jnp.concatenate(
        [h, jnp.ones((rows, 1), jnp.float32), jnp.zeros((rows, WPAD - D - 1), jnp.float32)],
        axis=1)


def _tables_body(x_ref, wcat_ref, t1_ref, t2_ref):
    h = jnp.dot(x_ref[...], wcat_ref[...], preferred_element_type=jnp.float32)
    t1_ref[...] = _pad_table(h[:, :D])
    t2_ref[...] = _pad_table(h[:, D:])


def _mid_body(sum1_ref, cnt1_ref, sum2_ref, cnt2_ref, xp_ref, xc_ref,
              w1r_ref, b1l_ref, w2r_ref, b2l_ref, w3l_ref,
              t3_ref, cust2_ref):
    mean1 = sum1_ref[...] / jnp.maximum(cnt1_ref[...], 1.0)
    px = jnp.maximum(
        mean1 + b1l_ref[...]
        + jnp.dot(xp_ref[...], w1r_ref[...], preferred_element_type=jnp.float32),
        0.0)
    h3 = jnp.dot(px, w3l_ref[...], preferred_element_type=jnp.float32)
    t3_ref[...] = _pad_table(h3)
    mean2 = sum2_ref[...] / jnp.maximum(cnt2_ref[...], 1.0)
    cust2_ref[...] = jnp.maximum(
        mean2 + b2l_ref[...]
        + jnp.dot(xc_ref[...], w2r_ref[...], preferred_element_type=jnp.float32),
        0.0)


def _final_body(sum3a_ref, cnt3a_ref, sum3b_ref, cnt3b_ref, cust2_ref,
                w3r_ref, b3l_ref, wlin_ref, blin_ref, out_ref):
    mean3 = ((sum3a_ref[...] + sum3b_ref[...])
             / jnp.maximum(cnt3a_ref[...] + cnt3b_ref[...], 1.0))
    c3 = jnp.maximum(
        mean3 + b3l_ref[...]
        + jnp.dot(cust2_ref[...], w3r_ref[...], preferred_element_type=jnp.float32),
        0.0)
    out_ref[...] = (jnp.dot(c3, wlin_ref[...], preferred_element_type=jnp.float32)
                    + blin_ref[...])


def _row_spec(width):
    return pl.BlockSpec((R, width), lambda i: (i, 0))


def _full_spec(shape):
    return pl.BlockSpec(shape, lambda i: tuple(0 for _ in shape))


def _split_acc(acc):
    return acc[:, :D], acc[:, D:D + 1]


def kernel(x_product, x_customer, edge_index_pp, edge_index_pc,
           W1l, b1l, W1r, W2l, b2l, W2r, W3l, b3l, W3r, Wlin, blin):
    E = edge_index_pp.shape[1]
    grid = (N // R,)

    wcat = jnp.concatenate([W1l, W2l], axis=1)
    t1, t2 = pl.pallas_call(
        _tables_body,
        grid=grid,
        in_specs=[_row_spec(D), _full_spec((D, 2 * D))],
        out_specs=[_row_spec(WPAD), _row_spec(WPAD)],
        out_shape=[jax.ShapeDtypeStruct((N, WPAD), jnp.float32)] * 2,
    )(x_product, wcat)

    n_chunks1 = E // (NSUB * K)
    nb1 = n_chunks1 // IB
    src_pp = edge_index_pp[0].reshape(NSUB, nb1, IB, K)
    dst_pp = edge_index_pp[1].reshape(NSUB, nb1, IB, K)
    src_pc = edge_index_pc[0].reshape(NSUB, nb1, IB, K)
    dst_pc = edge_index_pc[1].reshape(NSUB, nb1, IB, K)
    zeros_hbm = jnp.zeros((ROWS_PER_TILE, WPAD), jnp.float32)
    acc1, acc2 = _make_sc_segsum(n_chunks1)(
        t1, t2, src_pp, dst_pp, src_pc, dst_pc, zeros_hbm)

    sum1, cnt1 = _split_acc(acc1)
    sum2, cnt2 = _split_acc(acc2)
    t3, cust2 = pl.pallas_call(
        _mid_body,
        grid=grid,
        in_specs=[_row_spec(D), _row_spec(1), _row_spec(D), _row_spec(1),
                  _row_spec(D), _row_spec(D),
                  _full_spec((D, D)), _full_spec((1, D)),
                  _full_spec((D, D)), _full_spec((1, D)),
                  _full_spec((D, D))],
        out_specs=[_row_spec(WPAD), _row_spec(D)],
        out_shape=[jax.ShapeDtypeStruct((N, WPAD), jnp.float32),
                   jax.ShapeDtypeStruct((N, D), jnp.float32)],
    )(sum1, cnt1, sum2, cnt2, x_product, x_customer,
      W1r, b1l.reshape(1, D), W2r, b2l.reshape(1, D), W3l)

    n_chunks3 = E // (2 * NSUB * K)
    nb3 = n_chunks3 // IB
    src3 = edge_index_pc[0].reshape(2, NSUB, nb3, IB, K)
    dst3 = edge_index_pc[1].reshape(2, NSUB, nb3, IB, K)
    acc3a, acc3b = _make_sc_segsum(n_chunks3)(
        t3, t3, src3[0], dst3[0], src3[1], dst3[1], zeros_hbm)

    sum3a, cnt3a = _split_acc(acc3a)
    sum3b, cnt3b = _split_acc(acc3b)
    out = pl.pallas_call(
        _final_body,
        grid=grid,
        in_specs=[_row_spec(D), _row_spec(1), _row_spec(D), _row_spec(1),
                  _row_spec(D),
                  _full_spec((D, D)), _full_spec((1, D)),
                  _full_spec((D, D)), _full_spec((1, D))],
        out_specs=_row_spec(D),
        out_shape=jax.ShapeDtypeStruct((N, D), jnp.float32),
    )(sum3a, cnt3a, sum3b, cnt3b, cust2,
      W3r, b3l.reshape(1, D), Wlin, blin.reshape(1, D))

    return out
